# baseline (speedup 1.0000x reference)
import functools

import jax
import jax.numpy as jnp
from jax import lax
from jax.experimental import pallas as pl
from jax.experimental.pallas import tpu as pltpu
from jax.experimental.pallas import tpu_sc as plsc

HC = 128
K = 10
NUM_GRAPHS = 16
IN_FEAT = 39
EPS = 1e-5
SEG = 1024
KPAD = 16


def _pq_call(xp, wcat, b2d, n_pad):
    nblk = n_pad // SEG

    def body(x_ref, w_ref, b_ref, p_ref, q_ref):
        xw = jnp.dot(x_ref[...], w_ref[...], preferred_element_type=jnp.float32)
        p_ref[...] = xw[:, :HC] + b_ref[...]
        q_ref[...] = xw[:, HC:]

    return pl.pallas_call(
        body,
        grid=(nblk,),
        in_specs=[
            pl.BlockSpec((SEG, HC), lambda i: (i, 0)),
            pl.BlockSpec((HC, 2 * HC), lambda i: (0, 0)),
            pl.BlockSpec((1, HC), lambda i: (0, 0)),
        ],
        out_specs=[
            pl.BlockSpec((SEG, HC), lambda i: (i, 0)),
            pl.BlockSpec((SEG, HC), lambda i: (i, 0)),
        ],
        out_shape=[
            jax.ShapeDtypeStruct((n_pad, HC), jnp.float32),
            jax.ShapeDtypeStruct((n_pad, HC), jnp.float32),
        ],
    )(xp, wcat, b2d)


def _knn_call(starts, xp, n_pad):
    def body(starts_ref, x_ref, idx_ref):
        s = pl.program_id(0)
        start = starts_ref[s]
        size = starts_ref[s + 1] - start
        xs = x_ref[pl.ds(start, SEG), :]
        sq = jnp.sum(xs * xs, axis=1)
        g = lax.dot_general(xs, xs, (((1,), (1,)), ((), ())),
                            preferred_element_type=jnp.float32)
        d2 = sq[:, None] + sq[None, :] - 2.0 * g
        colio = lax.broadcasted_iota(jnp.int32, (SEG, SEG), 1)
        d2 = jnp.where(colio < size, d2, jnp.inf)
        laneio = lax.broadcasted_iota(jnp.int32, (SEG, KPAD), 1)
        idxacc = jnp.zeros((SEG, KPAD), jnp.int32)
        cur = d2
        for k in range(K):
            m = jnp.min(cur, axis=1)
            am = jnp.min(jnp.where(cur == m[:, None], colio, SEG), axis=1)
            idxacc = jnp.where(laneio == k, (am + start)[:, None], idxacc)
            cur = jnp.where(colio == am[:, None], jnp.inf, cur)
        idx_ref[pl.ds(start, SEG), :] = idxacc

    grid_spec = pltpu.PrefetchScalarGridSpec(
        num_scalar_prefetch=1,
        grid=(NUM_GRAPHS,),
        in_specs=[pl.BlockSpec((n_pad, HC), lambda s, starts: (0, 0))],
        out_specs=pl.BlockSpec((n_pad, KPAD), lambda s, starts: (0, 0)),
    )
    return pl.pallas_call(
        body,
        grid_spec=grid_spec,
        out_shape=jax.ShapeDtypeStruct((n_pad, KPAD), jnp.int32),
        compiler_params=pltpu.CompilerParams(
            dimension_semantics=("arbitrary",)),
    )(starts, xp)


def _sc_gather_max(q, idx_flat, n_pad):
    info = plsc.get_sparse_core_info()
    nc, ns = info.num_cores, info.num_subcores
    nw = nc * ns
    npw = n_pad // nw
    ch = 32
    nchunk = npw // ch
    rows = ch * K
    nvec = HC // 16
    splits = []
    off = 0
    while off < rows:
        w = min(128, rows - off)
        splits.append((off, w))
        off += w

    mesh = plsc.VectorSubcoreMesh(core_axis_name="c", subcore_axis_name="s")

    @functools.partial(
        pl.kernel,
        mesh=mesh,
        out_type=jax.ShapeDtypeStruct((n_pad, HC), jnp.float32),
        scratch_types=[
            pltpu.VMEM((rows,), jnp.int32),
            pltpu.VMEM((rows,), jnp.int32),
            pltpu.VMEM((rows, HC), jnp.float32),
            pltpu.VMEM((rows, HC), jnp.float32),
            pltpu.VMEM((ch, HC), jnp.float32),
            pltpu.VMEM((ch, HC), jnp.float32),
            pltpu.SemaphoreType.DMA,
            pltpu.SemaphoreType.DMA,
            pltpu.SemaphoreType.DMA,
        ],
    )
    def body(q_hbm, idx_hbm, out_hbm, idx0, idx1, rows0, rows1, m0, m1,
             semi, semg, semw):
        wid = lax.axis_index("s") * nc + lax.axis_index("c")
        idxb = [idx0, idx1]
        rowsb = [rows0, rows1]
        mb = [m0, m1]

        def stage_idx(i):
            nb = wid * npw + i * ch
            return pltpu.async_copy(
                idx_hbm.at[pl.ds(nb * K, rows)], idxb[i % 2], semi)

        def issue_gathers(i):
            return [
                pltpu.async_copy(
                    q_hbm.at[idxb[i % 2].at[pl.ds(o, w)]],
                    rowsb[i % 2].at[pl.ds(o, w)],
                    semg,
                )
                for o, w in splits
            ]

        def compute(i):
            rv = rowsb[i % 2]
            mv = mb[i % 2]

            @plsc.parallel_loop(0, ch, 1, unroll=4)
            def node_body(n):
                base = n * K
                for v in range(nvec):
                    sl = pl.ds(v * 16, 16)
                    acc = rv[base, sl]
                    for kk in range(1, K):
                        acc = jnp.maximum(acc, rv[base + kk, sl])
                    mv[n, sl] = acc
            nb = wid * npw + i * ch
            return pltpu.async_copy(mv, out_hbm.at[pl.ds(nb, ch)], semw)

        stage_idx(0).wait()
        gather_pend = issue_gathers(0)
        idx_pend = stage_idx(1) if nchunk > 1 else None
        write_pend = [None, None]
        for i in range(nchunk):
            for c in gather_pend:
                c.wait()
            if i + 1 < nchunk:
                idx_pend.wait()
                gather_pend = issue_gathers(i + 1)
                if i + 2 < nchunk:
                    idx_pend = stage_idx(i + 2)
            if write_pend[i % 2] is not None:
                write_pend[i % 2].wait()
            write_pend[i % 2] = compute(i)
        for wpend in write_pend:
            if wpend is not None:
                wpend.wait()

    return body(q, idx_flat)


def _combine_bn_call(p, m, g2d, be2d, n_real, n_pad, relu_after):
    def body(p_ref, m_ref, g_ref, be_ref, o_ref):
        h = jnp.maximum(p_ref[...] + m_ref[...], 0.0)
        rowio = lax.broadcasted_iota(jnp.int32, (n_pad, HC), 0)
        valid = rowio < n_real
        hm = jnp.where(valid, h, 0.0)
        mean = jnp.sum(hm, axis=0, keepdims=True) / n_real
        d = h - mean
        var = jnp.sum(jnp.where(valid, d * d, 0.0), axis=0,
                      keepdims=True) / n_real
        y = g_ref[...] * d / jnp.sqrt(var + EPS) + be_ref[...]
        if relu_after:
            y = jnp.maximum(y, 0.0)
        o_ref[...] = jnp.where(valid, y, 0.0)

    return pl.pallas_call(
        body,
        in_specs=[
            pl.BlockSpec((n_pad, HC), lambda: (0, 0)),
            pl.BlockSpec((n_pad, HC), lambda: (0, 0)),
            pl.BlockSpec((1, HC), lambda: (0, 0)),
            pl.BlockSpec((1, HC), lambda: (0, 0)),
        ],
        out_specs=pl.BlockSpec((n_pad, HC), lambda: (0, 0)),
        out_shape=jax.ShapeDtypeStruct((n_pad, HC), jnp.float32),
    )(p, m, g2d, be2d)


def _pool_head_call(h, batch2d, wl_pad, bl_pad, n_pad):
    def body(h_ref, b_ref, wl_ref, bl_ref, o_ref):
        seg = jnp.broadcast_to(b_ref[...], (NUM_GRAPHS, n_pad))
        sio = lax.broadcasted_iota(jnp.int32, (NUM_GRAPHS, n_pad), 0)
        onehot = (seg == sio).astype(jnp.float32)
        cnt = jnp.sum(onehot, axis=1)
        pooled = jnp.dot(onehot, h_ref[...],
                         preferred_element_type=jnp.float32)
        pooled = pooled / jnp.maximum(cnt, 1.0)[:, None]
        logits = jnp.dot(pooled, wl_ref[...],
                         preferred_element_type=jnp.float32) + bl_ref[...]
        clmask = lax.broadcasted_iota(jnp.int32, (NUM_GRAPHS, HC), 1) < 2
        z = jnp.where(clmask, logits, -jnp.inf)
        zmax = jnp.max(z, axis=1, keepdims=True)
        e = jnp.exp(z - zmax)
        o_ref[...] = e / jnp.sum(e, axis=1, keepdims=True)

    return pl.pallas_call(
        body,
        in_specs=[
            pl.BlockSpec((n_pad, HC), lambda: (0, 0)),
            pl.BlockSpec((1, n_pad), lambda: (0, 0)),
            pl.BlockSpec((HC, HC), lambda: (0, 0)),
            pl.BlockSpec((1, HC), lambda: (0, 0)),
        ],
        out_specs=pl.BlockSpec((NUM_GRAPHS, HC), lambda: (0, 0)),
        out_shape=jax.ShapeDtypeStruct((NUM_GRAPHS, HC), jnp.float32),
    )(h, batch2d, wl_pad, bl_pad)


def _split_w(w, d):
    wa, wb = w[:d], w[d:]
    wd = wa - wb
    pad = HC - d
    if pad:
        wd = jnp.pad(wd, ((0, pad), (0, 0)))
        wb = jnp.pad(wb, ((0, pad), (0, 0)))
    return jnp.concatenate([wd, wb], axis=1)


def _layer(xp, starts, wcat, b2d, g2d, be2d, n_real, n_pad, relu_after):
    p, q = _pq_call(xp, wcat, b2d, n_pad)
    idx16 = _knn_call(starts, xp, n_pad)
    idx_flat = jnp.clip(idx16[:, :K].reshape(-1), 0, n_pad - 1)
    m = _sc_gather_max(q, idx_flat, n_pad)
    return _combine_bn_call(p, m, g2d, be2d, n_real, n_pad, relu_after)


def kernel(x, edge_index, batch, W1, b1, W2, b2, W3, b3,
           g1, be1, g2, be2, g3, be3, Wl, bl):
    del edge_index
    n = x.shape[0]
    n_pad = ((n + SEG - 1) // SEG + 1) * SEG

    batch = batch.astype(jnp.int32)
    starts = jnp.searchsorted(
        batch, jnp.arange(NUM_GRAPHS + 1, dtype=jnp.int32)).astype(jnp.int32)

    xp = jnp.zeros((n_pad, HC), jnp.float32).at[:n, :IN_FEAT].set(x)
    batch2d = jnp.full((1, n_pad), NUM_GRAPHS, jnp.int32).at[0, :n].set(batch)

    wc1 = _split_w(W1, IN_FEAT)
    wc2 = _split_w(W2, HC)
    wc3 = _split_w(W3, HC)
    wl_pad = jnp.zeros((HC, HC), jnp.float32).at[:, :2].set(Wl)
    bl_pad = jnp.zeros((1, HC), jnp.float32).at[0, :2].set(bl)

    h = _layer(xp, starts, wc1, b1[None, :], g1[None, :], be1[None, :],
               n, n_pad, True)
    h = _layer(h, starts, wc2, b2[None, :], g2[None, :], be2[None, :],
               n, n_pad, True)
    h = _layer(h, starts, wc3, b3[None, :], g3[None, :], be3[None, :],
               n, n_pad, False)
    out = _pool_head_call(h, batch2d, wl_pad, bl_pad, n_pad)
    return out[:, :2]


# speedup vs baseline: 1.6580x; 1.6580x over previous
import functools

import jax
import jax.numpy as jnp
from jax import lax
from jax.experimental import pallas as pl
from jax.experimental.pallas import tpu as pltpu
from jax.experimental.pallas import tpu_sc as plsc

HC = 128
K = 10
NUM_GRAPHS = 16
IN_FEAT = 39
EPS = 1e-5
SEG = 1024
KPAD = 16


def _pq_call(xp, wcat, b2d, n_pad):
    nblk = n_pad // SEG

    def body(x_ref, w_ref, b_ref, p_ref, q_ref):
        xw = jnp.dot(x_ref[...], w_ref[...], preferred_element_type=jnp.float32)
        p_ref[...] = xw[:, :HC] + b_ref[...]
        q_ref[...] = xw[:, HC:]

    return pl.pallas_call(
        body,
        grid=(nblk,),
        in_specs=[
            pl.BlockSpec((SEG, HC), lambda i: (i, 0)),
            pl.BlockSpec((HC, 2 * HC), lambda i: (0, 0)),
            pl.BlockSpec((1, HC), lambda i: (0, 0)),
        ],
        out_specs=[
            pl.BlockSpec((SEG, HC), lambda i: (i, 0)),
            pl.BlockSpec((SEG, HC), lambda i: (i, 0)),
        ],
        out_shape=[
            jax.ShapeDtypeStruct((n_pad, HC), jnp.float32),
            jax.ShapeDtypeStruct((n_pad, HC), jnp.float32),
        ],
    )(xp, wcat, b2d)


def _knn_call(starts, xp, n_pad):
    def body(starts_ref, x_ref, idx_ref):
        s = pl.program_id(0)
        start = starts_ref[s]
        size = starts_ref[s + 1] - start
        xs = x_ref[pl.ds(start, SEG), :]
        sq = jnp.sum(xs * xs, axis=1)
        g = lax.dot_general(xs, xs, (((1,), (1,)), ((), ())),
                            preferred_element_type=jnp.float32)
        d2 = sq[:, None] + sq[None, :] - 2.0 * g
        colio = lax.broadcasted_iota(jnp.int32, (SEG, SEG), 1)
        d2 = jnp.where(colio < size, d2, jnp.inf)
        laneio = lax.broadcasted_iota(jnp.int32, (SEG, KPAD), 1)
        idxacc = jnp.zeros((SEG, KPAD), jnp.int32)
        cur = d2
        for k in range(K):
            m = jnp.min(cur, axis=1)
            am = jnp.min(jnp.where(cur == m[:, None], colio, SEG), axis=1)
            idxacc = jnp.where(laneio == k, (am + start)[:, None], idxacc)
            cur = jnp.where(colio == am[:, None], jnp.inf, cur)
        idx_ref[pl.ds(start, SEG), :] = idxacc

    grid_spec = pltpu.PrefetchScalarGridSpec(
        num_scalar_prefetch=1,
        grid=(NUM_GRAPHS,),
        in_specs=[pl.BlockSpec((n_pad, HC), lambda s, starts: (0, 0))],
        out_specs=pl.BlockSpec((n_pad, KPAD), lambda s, starts: (0, 0)),
    )
    return pl.pallas_call(
        body,
        grid_spec=grid_spec,
        out_shape=jax.ShapeDtypeStruct((n_pad, KPAD), jnp.int32),
        compiler_params=pltpu.CompilerParams(
            dimension_semantics=("arbitrary",)),
    )(starts, xp)


def _sc_gather_max(q, idx_flat, n_pad, n_tab):
    info = plsc.get_sparse_core_info()
    nc, ns = info.num_cores, info.num_subcores
    nw = nc * ns
    npw = n_pad // nw
    ch = 32
    nchunk = npw // ch
    rows = ch * K
    nvec = HC // 16
    splits = []
    off = 0
    while off < rows:
        w = min(128, rows - off)
        splits.append((off, w))
        off += w

    npc = n_tab // ns

    mesh = plsc.VectorSubcoreMesh(core_axis_name="c", subcore_axis_name="s")

    @functools.partial(
        pl.kernel,
        mesh=mesh,
        out_type=jax.ShapeDtypeStruct((n_pad, HC), jnp.float32),
        scratch_types=[
            pltpu.VMEM_SHARED((n_tab, HC), jnp.float32),
            pltpu.VMEM((rows,), jnp.int32),
            pltpu.VMEM((rows,), jnp.int32),
            pltpu.VMEM((rows, HC), jnp.float32),
            pltpu.VMEM((ch, HC), jnp.float32),
            pltpu.SemaphoreType.DMA,
            pltpu.SemaphoreType.DMA,
            pltpu.SemaphoreType.DMA,
        ],
    )
    def body(q_hbm, idx_hbm, out_hbm, q_sh, idx0, idx1, rows_v, m_v,
             semi, semg, semw):
        sid = lax.axis_index("s")
        wid = sid * nc + lax.axis_index("c")
        idxb = [idx0, idx1]

        stage = pltpu.async_copy(q_hbm.at[pl.ds(sid * npc, npc)],
                                 q_sh.at[pl.ds(sid * npc, npc)], semw)

        def stage_idx(i):
            nb = wid * npw + i * ch
            return pltpu.async_copy(
                idx_hbm.at[pl.ds(nb * K, rows)], idxb[i % 2], semi)

        idx_pend = stage_idx(0)
        stage.wait()
        plsc.subcore_barrier()

        for i in range(nchunk):
            idx_pend.wait()
            gathers = [
                pltpu.async_copy(
                    q_sh.at[idxb[i % 2].at[pl.ds(o, w)]],
                    rows_v.at[pl.ds(o, w)],
                    semg,
                )
                for o, w in splits
            ]
            if i + 1 < nchunk:
                idx_pend = stage_idx(i + 1)
            for c in gathers:
                c.wait()

            @plsc.parallel_loop(0, ch, 1, unroll=4)
            def node_body(n):
                base = n * K
                for v in range(nvec):
                    sl = pl.ds(v * 16, 16)
                    acc = rows_v[base, sl]
                    for kk in range(1, K):
                        acc = jnp.maximum(acc, rows_v[base + kk, sl])
                    m_v[n, sl] = acc

            nb = wid * npw + i * ch
            pltpu.sync_copy(m_v, out_hbm.at[pl.ds(nb, ch)])

    return body(q, idx_flat)


def _sc_linear_copy(q, idx_flat, n_pad):
    info = plsc.get_sparse_core_info()
    nc, ns = 1, info.num_subcores
    nw = nc * ns
    npw = n_pad // nw
    mesh = plsc.VectorSubcoreMesh(core_axis_name="c", subcore_axis_name="s",
                                  num_cores=nc)

    @functools.partial(
        pl.kernel,
        mesh=mesh,
        out_type=jax.ShapeDtypeStruct((n_pad, HC), jnp.float32),
        scratch_types=[
            pltpu.VMEM((npw, HC), jnp.float32),
        ],
    )
    def body(q_hbm, idx_hbm, out_hbm, buf):
        wid = lax.axis_index("s") * nc + lax.axis_index("c")
        nb = wid * npw
        pltpu.sync_copy(q_hbm.at[pl.ds(nb, npw)], buf)
        pltpu.sync_copy(buf, out_hbm.at[pl.ds(nb, npw)])

    return body(q, idx_flat)


def _combine_bn_call(p, m, g2d, be2d, n_real, n_pad, relu_after):
    def body(p_ref, m_ref, g_ref, be_ref, o_ref):
        h = jnp.maximum(p_ref[...] + m_ref[...], 0.0)
        rowio = lax.broadcasted_iota(jnp.int32, (n_pad, HC), 0)
        valid = rowio < n_real
        hm = jnp.where(valid, h, 0.0)
        mean = jnp.sum(hm, axis=0, keepdims=True) / n_real
        d = h - mean
        var = jnp.sum(jnp.where(valid, d * d, 0.0), axis=0,
                      keepdims=True) / n_real
        y = g_ref[...] * d / jnp.sqrt(var + EPS) + be_ref[...]
        if relu_after:
            y = jnp.maximum(y, 0.0)
        o_ref[...] = jnp.where(valid, y, 0.0)

    return pl.pallas_call(
        body,
        in_specs=[
            pl.BlockSpec((n_pad, HC), lambda: (0, 0)),
            pl.BlockSpec((n_pad, HC), lambda: (0, 0)),
            pl.BlockSpec((1, HC), lambda: (0, 0)),
            pl.BlockSpec((1, HC), lambda: (0, 0)),
        ],
        out_specs=pl.BlockSpec((n_pad, HC), lambda: (0, 0)),
        out_shape=jax.ShapeDtypeStruct((n_pad, HC), jnp.float32),
    )(p, m, g2d, be2d)


def _pool_head_call(h, batch2d, wl_pad, bl_pad, n_pad):
    def body(h_ref, b_ref, wl_ref, bl_ref, o_ref):
        seg = jnp.broadcast_to(b_ref[...], (NUM_GRAPHS, n_pad))
        sio = lax.broadcasted_iota(jnp.int32, (NUM_GRAPHS, n_pad), 0)
        onehot = (seg == sio).astype(jnp.float32)
        cnt = jnp.sum(onehot, axis=1)
        pooled = jnp.dot(onehot, h_ref[...],
                         preferred_element_type=jnp.float32)
        pooled = pooled / jnp.maximum(cnt, 1.0)[:, None]
        logits = jnp.dot(pooled, wl_ref[...],
                         preferred_element_type=jnp.float32) + bl_ref[...]
        clmask = lax.broadcasted_iota(jnp.int32, (NUM_GRAPHS, HC), 1) < 2
        z = jnp.where(clmask, logits, -jnp.inf)
        zmax = jnp.max(z, axis=1, keepdims=True)
        e = jnp.exp(z - zmax)
        o_ref[...] = e / jnp.sum(e, axis=1, keepdims=True)

    return pl.pallas_call(
        body,
        in_specs=[
            pl.BlockSpec((n_pad, HC), lambda: (0, 0)),
            pl.BlockSpec((1, n_pad), lambda: (0, 0)),
            pl.BlockSpec((HC, HC), lambda: (0, 0)),
            pl.BlockSpec((1, HC), lambda: (0, 0)),
        ],
        out_specs=pl.BlockSpec((NUM_GRAPHS, HC), lambda: (0, 0)),
        out_shape=jax.ShapeDtypeStruct((NUM_GRAPHS, HC), jnp.float32),
    )(h, batch2d, wl_pad, bl_pad)


def _split_w(w, d):
    wa, wb = w[:d], w[d:]
    wd = wa - wb
    pad = HC - d
    if pad:
        wd = jnp.pad(wd, ((0, pad), (0, 0)))
        wb = jnp.pad(wb, ((0, pad), (0, 0)))
    return jnp.concatenate([wd, wb], axis=1)


def _layer(xp, starts, wcat, b2d, g2d, be2d, n_real, n_pad, relu_after):
    p, q = _pq_call(xp, wcat, b2d, n_pad)
    idx16 = _knn_call(starts, xp, n_pad)
    idx_flat = jnp.clip(idx16[:, :K].reshape(-1), 0, n_real - 1)
    n_tab = ((n_real + 127) // 128) * 128
    m = _sc_gather_max(q, idx_flat, n_pad, n_tab)
    return _combine_bn_call(p, m, g2d, be2d, n_real, n_pad, relu_after)


def kernel(x, edge_index, batch, W1, b1, W2, b2, W3, b3,
           g1, be1, g2, be2, g3, be3, Wl, bl):
    del edge_index
    n = x.shape[0]
    n_pad = ((n + SEG - 1) // SEG + 1) * SEG

    batch = batch.astype(jnp.int32)
    starts = jnp.searchsorted(
        batch, jnp.arange(NUM_GRAPHS + 1, dtype=jnp.int32)).astype(jnp.int32)

    xp = jnp.zeros((n_pad, HC), jnp.float32).at[:n, :IN_FEAT].set(x)
    batch2d = jnp.full((1, n_pad), NUM_GRAPHS, jnp.int32).at[0, :n].set(batch)

    wc1 = _split_w(W1, IN_FEAT)
    wc2 = _split_w(W2, HC)
    wc3 = _split_w(W3, HC)
    wl_pad = jnp.zeros((HC, HC), jnp.float32).at[:, :2].set(Wl)
    bl_pad = jnp.zeros((1, HC), jnp.float32).at[0, :2].set(bl)

    h = _layer(xp, starts, wc1, b1[None, :], g1[None, :], be1[None, :],
               n, n_pad, True)
    h = _layer(h, starts, wc2, b2[None, :], g2[None, :], be2[None, :],
               n, n_pad, True)
    h = _layer(h, starts, wc3, b3[None, :], g3[None, :], be3[None, :],
               n, n_pad, False)
    out = _pool_head_call(h, batch2d, wl_pad, bl_pad, n_pad)
    return out[:, :2]
